# Initial kernel scaffold; baseline (speedup 1.0000x reference)
import functools

import jax
import jax.numpy as jnp
from jax import lax
from jax.experimental import pallas as pl
from jax.experimental.pallas import tpu as pltpu
from jax.experimental.pallas import tpu_sc as plsc

N = 10000
E = 320000
R = 8
B = 30
D = 128
H1 = 128
H2 = 128
NR = N * R

NC = 2
NS = 16
NW = NC * NS
EW = E // NW
K = 80
CH = EW // K

NB = 10
BN = N // NB

_mesh = plsc.VectorSubcoreMesh(
    core_axis_name="c", subcore_axis_name="s", num_cores=NC, num_subcores=NS)


@functools.partial(
    pl.kernel,
    out_type=jax.ShapeDtypeStruct((NC, NR), jnp.float32),
    mesh=_mesh,
    scratch_types=[
        pltpu.VMEM((K,), jnp.int32),
        pltpu.VMEM((K,), jnp.int32),
        pltpu.VMEM((K,), jnp.int32),
        pltpu.VMEM((K,), jnp.float32),
        pltpu.VMEM((NR // NS,), jnp.float32),
        pltpu.VMEM_SHARED((NR,), jnp.float32),
    ],
)
def _count_kernel(dst_hbm, et_hbm, out_hbm, dst_v, et_v, seg_v, ones_v,
                  z_v, cnt_sh):
    c = lax.axis_index("c")
    s = lax.axis_index("s")

    def zfill(i, _):
        z_v[pl.ds(i * 16, 16)] = jnp.zeros((16,), jnp.float32)
        return 0
    lax.fori_loop(0, (NR // NS) // 16, zfill, 0)
    for g in range(K // 16):
        ones_v[pl.ds(g * 16, 16)] = jnp.ones((16,), jnp.float32)
    pltpu.sync_copy(z_v, cnt_sh.at[pl.ds(s * (NR // NS), NR // NS)])
    plsc.subcore_barrier()

    base = (c * NS + s) * EW

    def body(i, _):
        off = base + i * K
        pltpu.sync_copy(dst_hbm.at[pl.ds(off, K)], dst_v)
        pltpu.sync_copy(et_hbm.at[pl.ds(off, K)], et_v)
        for g in range(K // 16):
            sl = pl.ds(g * 16, 16)
            seg_v[sl] = dst_v[sl] * R + et_v[sl]
        pltpu.sync_copy(ones_v, cnt_sh.at[seg_v], add=True)
        return 0
    lax.fori_loop(0, CH, body, 0)

    plsc.subcore_barrier()

    @pl.when(s == 0)
    def _():
        pltpu.sync_copy(cnt_sh, out_hbm.at[c])


def _bcast16(vec, j):
    return jnp.take(vec, jnp.full((16,), j, jnp.int32),
                    mode="promise_in_bounds")


@functools.partial(
    pl.kernel,
    out_type=jax.ShapeDtypeStruct((NC, N, D), jnp.float32),
    mesh=_mesh,
    scratch_types=[
        pltpu.VMEM((K,), jnp.int32),
        pltpu.VMEM((K,), jnp.int32),
        pltpu.VMEM((K,), jnp.int32),
        pltpu.VMEM((K,), jnp.int32),
        pltpu.VMEM((K,), jnp.int32),
        pltpu.VMEM((NR,), jnp.float32),
        pltpu.VMEM((K, D), jnp.float32),
        pltpu.VMEM_SHARED((N, D), jnp.float32),
        pltpu.SemaphoreType.DMA,
    ],
)
def _edge_pass_a(src_hbm, dst_hbm, et_hbm, h_hbm, inv_hbm, zeros_hbm,
                 out_hbm, src_v, dst_v, et_v, ridx_v, seg_v, inv_v, rows_v,
                 acc_sh, sem):
    c = lax.axis_index("c")
    s = lax.axis_index("s")

    @pl.when(s == 0)
    def _():
        pltpu.sync_copy(zeros_hbm, acc_sh)
    pltpu.sync_copy(inv_hbm, inv_v)
    plsc.subcore_barrier()

    base = (c * NS + s) * EW

    def body(i, _):
        off = base + i * K
        pltpu.sync_copy(src_hbm.at[pl.ds(off, K)], src_v)
        pltpu.sync_copy(dst_hbm.at[pl.ds(off, K)], dst_v)
        pltpu.sync_copy(et_hbm.at[pl.ds(off, K)], et_v)
        for g in range(K // 16):
            sl = pl.ds(g * 16, 16)
            ridx_v[sl] = et_v[sl] * N + src_v[sl]
            seg_v[sl] = dst_v[sl] * R + et_v[sl]
        pltpu.async_copy(h_hbm.at[ridx_v], rows_v, sem).wait()
        for g in range(K // 16):
            w16 = plsc.load_gather(inv_v, [seg_v[pl.ds(g * 16, 16)]])
            for j in range(16):
                wb = _bcast16(w16, j)
                row = g * 16 + j
                for col in range(D // 16):
                    cs = pl.ds(col * 16, 16)
                    rows_v[row, cs] = rows_v[row, cs] * wb
        pltpu.sync_copy(rows_v, acc_sh.at[dst_v], add=True)
        return 0
    lax.fori_loop(0, CH, body, 0)

    plsc.subcore_barrier()

    @pl.when(s == 0)
    def _():
        pltpu.sync_copy(acc_sh, out_hbm.at[c])


@functools.partial(
    pl.kernel,
    out_type=jax.ShapeDtypeStruct((NC, N, D), jnp.float32),
    mesh=_mesh,
    scratch_types=[
        pltpu.VMEM((K,), jnp.int32),
        pltpu.VMEM((K,), jnp.int32),
        pltpu.VMEM((K,), jnp.float32),
        pltpu.VMEM((K, D), jnp.float32),
        pltpu.VMEM_SHARED((N, D), jnp.float32),
        pltpu.SemaphoreType.DMA,
    ],
)
def _edge_pass_b(src_hbm, dst_hbm, w_hbm, x1_hbm, zeros_hbm, out_hbm,
                 src_v, dst_v, w_v, rows_v, acc_sh, sem):
    c = lax.axis_index("c")
    s = lax.axis_index("s")

    @pl.when(s == 0)
    def _():
        pltpu.sync_copy(zeros_hbm, acc_sh)
    plsc.subcore_barrier()

    base = (c * NS + s) * EW

    def body(i, _):
        off = base + i * K
        pltpu.sync_copy(src_hbm.at[pl.ds(off, K)], src_v)
        pltpu.sync_copy(dst_hbm.at[pl.ds(off, K)], dst_v)
        pltpu.sync_copy(w_hbm.at[pl.ds(off, K)], w_v)
        pltpu.async_copy(x1_hbm.at[src_v], rows_v, sem).wait()
        for g in range(K // 16):
            w16 = w_v[pl.ds(g * 16, 16)]
            for j in range(16):
                wb = _bcast16(w16, j)
                row = g * 16 + j
                for col in range(D // 16):
                    cs = pl.ds(col * 16, 16)
                    rows_v[row, cs] = rows_v[row, cs] * wb
        pltpu.sync_copy(rows_v, acc_sh.at[dst_v], add=True)
        return 0
    lax.fori_loop(0, CH, body, 0)

    plsc.subcore_barrier()

    @pl.when(s == 0)
    def _():
        pltpu.sync_copy(acc_sh, out_hbm.at[c])


def _w_body(comp_ref, basis_ref, w_ref):
    r = pl.program_id(0)
    acc = basis_ref[0] * comp_ref[r, 0]
    for b in range(1, B):
        acc = acc + basis_ref[b] * comp_ref[r, b]
    w_ref[0] = acc


def _w_kernel(comp, basis):
    return pl.pallas_call(
        _w_body,
        grid=(R,),
        in_specs=[
            pl.BlockSpec(memory_space=pltpu.SMEM),
            pl.BlockSpec((B, D, H1), lambda r: (0, 0, 0)),
        ],
        out_specs=pl.BlockSpec((1, D, H1), lambda r: (r, 0, 0)),
        out_shape=jax.ShapeDtypeStruct((R, D, H1), jnp.float32),
    )(comp, basis)


def _inv_body(cnt_ref, inv_ref):
    cnt = cnt_ref[0] + cnt_ref[1]
    inv_ref[...] = jnp.where(cnt > 0.0, 1.0 / jnp.maximum(cnt, 1.0), 0.0)


def _inv_kernel(cnt2):
    return pl.pallas_call(
        _inv_body,
        out_shape=jax.ShapeDtypeStruct((NR // 128, 128), jnp.float32),
    )(cnt2)


def _h_body(x_ref, w_ref, h_ref):
    h_ref[...] = jnp.dot(x_ref[...], w_ref[0],
                         preferred_element_type=jnp.float32)


def _h_kernel(x, w):
    return pl.pallas_call(
        _h_body,
        grid=(NB, R),
        in_specs=[
            pl.BlockSpec((BN, D), lambda i, r: (i, 0)),
            pl.BlockSpec((1, D, H1), lambda i, r: (r, 0, 0)),
        ],
        out_specs=pl.BlockSpec((BN, H1), lambda i, r: (r * NB + i, 0)),
        out_shape=jax.ShapeDtypeStruct((R * N, H1), jnp.float32),
    )(x, w)


def _x1_body(agg_ref, x_ref, rw_ref, b_ref, x1_ref):
    x1_ref[...] = (agg_ref[0] + agg_ref[1]
                   + jnp.dot(x_ref[...], rw_ref[...],
                             preferred_element_type=jnp.float32)
                   + b_ref[...])


def _x1_kernel(agg2, x, root_w, bias1):
    return pl.pallas_call(
        _x1_body,
        grid=(NB,),
        in_specs=[
            pl.BlockSpec((NC, BN, H1), lambda i: (0, i, 0)),
            pl.BlockSpec((BN, D), lambda i: (i, 0)),
            pl.BlockSpec((D, H1), lambda i: (0, 0)),
            pl.BlockSpec((1, H1), lambda i: (0, 0)),
        ],
        out_specs=pl.BlockSpec((BN, H1), lambda i: (i, 0)),
        out_shape=jax.ShapeDtypeStruct((N, H1), jnp.float32),
    )(agg2, x, root_w, bias1)


def _out_body(agg_ref, x1_ref, wr_ref, wr2_ref, b_ref, o_ref):
    o_ref[...] = (jnp.dot(agg_ref[0] + agg_ref[1], wr_ref[...],
                          preferred_element_type=jnp.float32)
                  + jnp.dot(x1_ref[...], wr2_ref[...],
                            preferred_element_type=jnp.float32)
                  + b_ref[...])


def _out_kernel(agg2, x1, w_rel, w_root2, bias2):
    return pl.pallas_call(
        _out_body,
        grid=(NB,),
        in_specs=[
            pl.BlockSpec((NC, BN, H1), lambda i: (0, i, 0)),
            pl.BlockSpec((BN, H1), lambda i: (i, 0)),
            pl.BlockSpec((H1, H2), lambda i: (0, 0)),
            pl.BlockSpec((H1, H2), lambda i: (0, 0)),
            pl.BlockSpec((1, H2), lambda i: (0, 0)),
        ],
        out_specs=pl.BlockSpec((BN, H2), lambda i: (i, 0)),
        out_shape=jax.ShapeDtypeStruct((N, H2), jnp.float32),
    )(agg2, x1, w_rel, w_root2, bias2)


def kernel(node_features, edge_index, edge_norm, edge_type, basis, comp,
           root_w, bias1, w_rel, w_root2, bias2):
    src = edge_index[0]
    dst = edge_index[1]
    et = edge_type
    zeros_nd = jnp.zeros((N, D), jnp.float32)

    cnt2 = _count_kernel(dst, et)
    inv = _inv_kernel(cnt2.reshape(NC, NR // 128, 128))
    w_all = _w_kernel(comp, basis)
    h = _h_kernel(node_features, w_all)
    agg = _edge_pass_a(src, dst, et, h, inv.reshape(NR), zeros_nd)
    x1 = _x1_kernel(agg, node_features, root_w, bias1.reshape(1, H1))
    gg2 = _edge_pass_b(src, dst, edge_norm, x1, zeros_nd)
    return _out_kernel(gg2, x1, w_rel, w_root2, bias2.reshape(1, H2))


# speedup vs baseline: 7.3664x; 7.3664x over previous
import functools

import jax
import jax.numpy as jnp
from jax import lax
from jax.experimental import pallas as pl
from jax.experimental.pallas import tpu as pltpu
from jax.experimental.pallas import tpu_sc as plsc

N = 10000
E = 320000
R = 8
B = 30
D = 128
H1 = 128
H2 = 128
NR = N * R

NC = 2
NS = 16
NW = NC * NS
EW = E // NW
K = 80
CH = EW // K

NB = 10
BN = N // NB

_mesh = plsc.VectorSubcoreMesh(
    core_axis_name="c", subcore_axis_name="s", num_cores=NC, num_subcores=NS)
_sc_params = pltpu.CompilerParams(needs_layout_passes=False)


ET2 = E // NS
CH2 = ET2 // K
SEG_T = NR // NS


@functools.partial(
    pl.kernel,
    out_type=jax.ShapeDtypeStruct((E,), jnp.float32),
    mesh=_mesh,
    compiler_params=_sc_params,
    scratch_types=[
        pltpu.VMEM((K,), jnp.int32),
        pltpu.VMEM((K,), jnp.int32),
        pltpu.VMEM((K,), jnp.int32),
        pltpu.VMEM((K,), jnp.float32),
        pltpu.VMEM((K,), jnp.float32),
        pltpu.VMEM((SEG_T + 8,), jnp.float32),
        pltpu.VMEM_SHARED((NR,), jnp.float32),
        pltpu.SemaphoreType.DMA,
    ],
)
def _weights_kernel(dst_hbm, et_hbm, w_hbm, dst_v, et_v, seg_v, w_v,
                    ones_v, cb_v, cnt_sh, sem):
    c = lax.axis_index("c")
    s = lax.axis_index("s")

    def zfill(i, _):
        cb_v[pl.ds(i * 16, 16)] = jnp.zeros((16,), jnp.float32)
        return 0
    lax.fori_loop(0, (SEG_T + 8) // 16, zfill, 0)
    for g in range(K // 16):
        ones_v[pl.ds(g * 16, 16)] = jnp.ones((16,), jnp.float32)
    pltpu.sync_copy(cb_v.at[pl.ds(0, SEG_T)],
                    cnt_sh.at[pl.ds(s * SEG_T, SEG_T)])
    plsc.subcore_barrier()

    def count_body(i, _):
        off = s * ET2 + i * K
        pltpu.sync_copy(dst_hbm.at[pl.ds(off, K)], dst_v)
        pltpu.sync_copy(et_hbm.at[pl.ds(off, K)], et_v)
        for g in range(K // 16):
            sl = pl.ds(g * 16, 16)
            seg_v[sl] = dst_v[sl] * R + et_v[sl]
        pltpu.sync_copy(ones_v, cnt_sh.at[seg_v], add=True)
        return 0
    lax.fori_loop(0, CH2, count_body, 0)
    plsc.subcore_barrier()

    pltpu.sync_copy(cnt_sh.at[pl.ds(s * SEG_T, SEG_T)],
                    cb_v.at[pl.ds(0, SEG_T)])

    def inv_body(i, _):
        sl = pl.ds(i * 16, 16)
        v = cb_v[sl]
        cb_v[sl] = jnp.where(v > 0.0, 1.0 / jnp.maximum(v, 1.0), 0.0)
        return 0
    lax.fori_loop(0, (SEG_T + 8) // 16, inv_body, 0)
    pltpu.sync_copy(cb_v.at[pl.ds(0, SEG_T)],
                    cnt_sh.at[pl.ds(s * SEG_T, SEG_T)])
    plsc.subcore_barrier()

    base = (c * NS + s) * EW

    def w_body(i, _):
        off = base + i * K
        pltpu.sync_copy(dst_hbm.at[pl.ds(off, K)], dst_v)
        pltpu.sync_copy(et_hbm.at[pl.ds(off, K)], et_v)
        for g in range(K // 16):
            sl = pl.ds(g * 16, 16)
            seg_v[sl] = dst_v[sl] * R + et_v[sl]
        pltpu.async_copy(cnt_sh.at[seg_v], w_v, sem).wait()
        pltpu.sync_copy(w_v, w_hbm.at[pl.ds(off, K)])
        return 0
    lax.fori_loop(0, CH, w_body, 0)


def _bcast16(vec, j):
    return jnp.take_along_axis(vec, jnp.full((16,), j, jnp.int32), axis=0)


def _scale_rows(rows_v, w_v):
    for g in range(K // 16):
        w16 = w_v[pl.ds(g * 16, 16)]
        for j in range(16):
            wb = _bcast16(w16, j)
            row = g * 16 + j
            for col in range(D // 16):
                cs = pl.ds(col * 16, 16)
                rows_v[row, cs] = rows_v[row, cs] * wb


def _make_edge_pass(relation_indexed):
    scratch = [
        pltpu.VMEM((K,), jnp.int32),
        pltpu.VMEM((K,), jnp.int32),
        pltpu.VMEM((K,), jnp.float32),
        pltpu.VMEM((K, D), jnp.float32),
        pltpu.VMEM_SHARED((N, D), jnp.float32),
        pltpu.SemaphoreType.DMA,
    ]
    if relation_indexed:
        scratch = [pltpu.VMEM((K,), jnp.int32),
                   pltpu.VMEM((K,), jnp.int32)] + scratch

    def outer(src_hbm, dst_hbm, et_hbm, w_hbm, table_hbm, zeros_hbm,
              out_hbm, et_v, ridx_v, src_v, dst_v, w_v, rows_v, acc_sh,
              sem):
        c = lax.axis_index("c")
        s = lax.axis_index("s")

        @pl.when(s == 0)
        def _():
            pltpu.sync_copy(zeros_hbm, acc_sh)
        plsc.subcore_barrier()

        base = (c * NS + s) * EW

        def body(i, _):
            off = base + i * K
            pltpu.sync_copy(src_hbm.at[pl.ds(off, K)], src_v)
            pltpu.sync_copy(dst_hbm.at[pl.ds(off, K)], dst_v)
            pltpu.sync_copy(w_hbm.at[pl.ds(off, K)], w_v)
            if relation_indexed:
                pltpu.sync_copy(et_hbm.at[pl.ds(off, K)], et_v)
                for g in range(K // 16):
                    sl = pl.ds(g * 16, 16)
                    ridx_v[sl] = et_v[sl] * N + src_v[sl]
                pltpu.async_copy(table_hbm.at[ridx_v], rows_v, sem).wait()
            else:
                pltpu.async_copy(table_hbm.at[src_v], rows_v, sem).wait()
            _scale_rows(rows_v, w_v)
            pltpu.sync_copy(rows_v, acc_sh.at[dst_v], add=True)
            return 0
        lax.fori_loop(0, CH, body, 0)

        plsc.subcore_barrier()

        @pl.when(s == 0)
        def _():
            pltpu.sync_copy(acc_sh, out_hbm.at[c])

    if relation_indexed:
        def body_fn(src_hbm, dst_hbm, et_hbm, w_hbm, table_hbm, zeros_hbm,
                    out_hbm, et_v, ridx_v, src_v, dst_v, w_v, rows_v,
                    acc_sh, sem):
            outer(src_hbm, dst_hbm, et_hbm, w_hbm, table_hbm, zeros_hbm,
                  out_hbm, et_v, ridx_v, src_v, dst_v, w_v, rows_v, acc_sh,
                  sem)
    else:
        def body_fn(src_hbm, dst_hbm, w_hbm, table_hbm, zeros_hbm, out_hbm,
                    src_v, dst_v, w_v, rows_v, acc_sh, sem):
            outer(src_hbm, dst_hbm, None, w_hbm, table_hbm, zeros_hbm,
                  out_hbm, None, None, src_v, dst_v, w_v, rows_v, acc_sh,
                  sem)

    return pl.kernel(
        body_fn,
        out_type=jax.ShapeDtypeStruct((NC, N, D), jnp.float32),
        mesh=_mesh,
        compiler_params=_sc_params,
        scratch_types=scratch,
    )


_edge_pass_a = _make_edge_pass(True)
_edge_pass_b = _make_edge_pass(False)


def _w_body(comp_ref, basis_ref, w_ref):
    r = pl.program_id(0)
    acc = basis_ref[0] * comp_ref[r, 0]
    for b in range(1, B):
        acc = acc + basis_ref[b] * comp_ref[r, b]
    w_ref[0] = acc


def _w_kernel(comp, basis):
    return pl.pallas_call(
        _w_body,
        grid=(R,),
        in_specs=[
            pl.BlockSpec(memory_space=pltpu.SMEM),
            pl.BlockSpec((B, D, H1), lambda r: (0, 0, 0)),
        ],
        out_specs=pl.BlockSpec((1, D, H1), lambda r: (r, 0, 0)),
        out_shape=jax.ShapeDtypeStruct((R, D, H1), jnp.float32),
    )(comp, basis)


def _h_body(x_ref, w_ref, h_ref):
    h_ref[...] = jnp.dot(x_ref[...], w_ref[0],
                         preferred_element_type=jnp.float32)


def _h_kernel(x, w):
    return pl.pallas_call(
        _h_body,
        grid=(NB, R),
        in_specs=[
            pl.BlockSpec((BN, D), lambda i, r: (i, 0)),
            pl.BlockSpec((1, D, H1), lambda i, r: (r, 0, 0)),
        ],
        out_specs=pl.BlockSpec((BN, H1), lambda i, r: (r * NB + i, 0)),
        out_shape=jax.ShapeDtypeStruct((R * N, H1), jnp.float32),
    )(x, w)


def _x1_body(agg_ref, x_ref, rw_ref, b_ref, x1_ref):
    x1_ref[...] = (agg_ref[0] + agg_ref[1]
                   + jnp.dot(x_ref[...], rw_ref[...],
                             preferred_element_type=jnp.float32)
                   + b_ref[...])


def _x1_kernel(agg2, x, root_w, bias1):
    return pl.pallas_call(
        _x1_body,
        grid=(NB,),
        in_specs=[
            pl.BlockSpec((NC, BN, H1), lambda i: (0, i, 0)),
            pl.BlockSpec((BN, D), lambda i: (i, 0)),
            pl.BlockSpec((D, H1), lambda i: (0, 0)),
            pl.BlockSpec((1, H1), lambda i: (0, 0)),
        ],
        out_specs=pl.BlockSpec((BN, H1), lambda i: (i, 0)),
        out_shape=jax.ShapeDtypeStruct((N, H1), jnp.float32),
    )(agg2, x, root_w, bias1)


def _out_body(agg_ref, x1_ref, wr_ref, wr2_ref, b_ref, o_ref):
    o_ref[...] = (jnp.dot(agg_ref[0] + agg_ref[1], wr_ref[...],
                          preferred_element_type=jnp.float32)
                  + jnp.dot(x1_ref[...], wr2_ref[...],
                            preferred_element_type=jnp.float32)
                  + b_ref[...])


def _out_kernel(agg2, x1, w_rel, w_root2, bias2):
    return pl.pallas_call(
        _out_body,
        grid=(NB,),
        in_specs=[
            pl.BlockSpec((NC, BN, H1), lambda i: (0, i, 0)),
            pl.BlockSpec((BN, H1), lambda i: (i, 0)),
            pl.BlockSpec((H1, H2), lambda i: (0, 0)),
            pl.BlockSpec((H1, H2), lambda i: (0, 0)),
            pl.BlockSpec((1, H2), lambda i: (0, 0)),
        ],
        out_specs=pl.BlockSpec((BN, H2), lambda i: (i, 0)),
        out_shape=jax.ShapeDtypeStruct((N, H2), jnp.float32),
    )(agg2, x1, w_rel, w_root2, bias2)


def kernel(node_features, edge_index, edge_norm, edge_type, basis, comp,
           root_w, bias1, w_rel, w_root2, bias2):
    src = edge_index[0]
    dst = edge_index[1]
    et = edge_type
    zeros_nd = jnp.zeros((N, D), jnp.float32)

    w_edge = _weights_kernel(dst, et)
    w_all = _w_kernel(comp, basis)
    h = _h_kernel(node_features, w_all)
    agg = _edge_pass_a(src, dst, et, w_edge, h, zeros_nd)
    x1 = _x1_kernel(agg, node_features, root_w, bias1.reshape(1, H1))
    gg2 = _edge_pass_b(src, dst, edge_norm, x1, zeros_nd)
    return _out_kernel(gg2, x1, w_rel, w_root2, bias2.reshape(1, H2))


# speedup vs baseline: 9.4180x; 1.2785x over previous
import functools

import jax
import jax.numpy as jnp
from jax import lax
from jax.experimental import pallas as pl
from jax.experimental.pallas import tpu as pltpu
from jax.experimental.pallas import tpu_sc as plsc

N = 10000
E = 320000
R = 8
B = 30
D = 128
H1 = 128
H2 = 128
NR = N * R

NC = 2
NS = 16
NW = NC * NS
EW = E // NW
K = 80
CH = EW // K

NB = 10
BN = N // NB

_PERM = [(t // 32) * 32 + (t % 2) * 16 + (t % 32) // 2 for t in range(H1)]

_mesh = plsc.VectorSubcoreMesh(
    core_axis_name="c", subcore_axis_name="s", num_cores=NC, num_subcores=NS)
_sc_params = pltpu.CompilerParams(needs_layout_passes=False)
_sc_params_edge = pltpu.CompilerParams(needs_layout_passes=False,
                                       use_tc_tiling_on_sc=False)


EB = 2000
CB = EB // K
NBLK = EW // EB
ET2 = E // NS
NBLK2 = ET2 // EB
SEG_T = NR // NS


def _bcast16(vec, j):
    return jnp.take_along_axis(vec, jnp.full((16,), j, jnp.int32), axis=0)


def _scale_chunk_bf(w_v, woff, bufbf, sbuf):
    def scale_g(g, _):
        w16 = w_v[pl.ds(woff + g * 16, 16)]

        def scale_j4(j4, _):
            for u in range(4):
                jj = j4 * 4 + u
                wb = _bcast16(w16, jj)
                row = g * 16 + jj
                for grp in range(D // 32):
                    rg = plsc.bitcast(bufbf[row, pl.ds(grp * 16, 16)],
                                      jnp.bfloat16)
                    lo, hi = plsc.unpack(
                        rg, format=plsc.PackFormat.INTERLEAVED)
                    sbuf[row, pl.ds(grp * 32, 16)] = lo * wb
                    sbuf[row, pl.ds(grp * 32 + 16, 16)] = hi * wb
            return 0
        lax.fori_loop(0, 4, scale_j4, 0)
        return 0
    lax.fori_loop(0, K // 16, scale_g, 0)


@functools.partial(
    pl.kernel,
    out_type=jax.ShapeDtypeStruct((NR,), jnp.float32),
    mesh=_mesh,
    compiler_params=_sc_params,
    scratch_types=[
        pltpu.VMEM((EB,), jnp.int32),
        pltpu.VMEM((EB,), jnp.int32),
        pltpu.VMEM((EB,), jnp.int32),
        pltpu.VMEM((K,), jnp.float32),
        pltpu.VMEM((SEG_T + 8,), jnp.float32),
        pltpu.VMEM_SHARED((NR,), jnp.float32),
        pltpu.SemaphoreType.DMA,
    ],
)
def _inv_table_kernel(dst_hbm, et_hbm, inv_hbm, dst_v, et_v, seg_v, ones_v,
                      cb_v, cnt_sh, sem):
    c = lax.axis_index("c")
    s = lax.axis_index("s")

    def zfill(i, _):
        cb_v[pl.ds(i * 16, 16)] = jnp.zeros((16,), jnp.float32)
        return 0
    lax.fori_loop(0, (SEG_T + 8) // 16, zfill, 0)
    for g in range(K // 16):
        ones_v[pl.ds(g * 16, 16)] = jnp.ones((16,), jnp.float32)
    pltpu.sync_copy(cb_v.at[pl.ds(0, SEG_T)],
                    cnt_sh.at[pl.ds(s * SEG_T, SEG_T)])
    plsc.subcore_barrier()

    def count_block(b, _):
        off = s * ET2 + b * EB
        pltpu.sync_copy(dst_hbm.at[pl.ds(off, EB)], dst_v)
        pltpu.sync_copy(et_hbm.at[pl.ds(off, EB)], et_v)

        def seg_i(i, _):
            sl = pl.ds(i * 16, 16)
            seg_v[sl] = dst_v[sl] * R + et_v[sl]
            return 0
        lax.fori_loop(0, EB // 16, seg_i, 0)

        def fire(q, _):
            pltpu.async_copy(ones_v, cnt_sh.at[seg_v.at[pl.ds(q * K, K)]],
                             sem, add=True)
            return 0
        lax.fori_loop(0, CB, fire, 0)

        def drain(q, _):
            pltpu.make_async_copy(ones_v, cnt_sh.at[seg_v.at[pl.ds(0, K)]],
                                  sem).wait()
            return 0
        lax.fori_loop(0, CB, drain, 0)
        return 0
    lax.fori_loop(0, NBLK2, count_block, 0)
    plsc.subcore_barrier()

    @pl.when(c == 0)
    def _():
        pltpu.sync_copy(cnt_sh.at[pl.ds(s * SEG_T, SEG_T)],
                        cb_v.at[pl.ds(0, SEG_T)])

        def inv_body(i, _):
            sl = pl.ds(i * 16, 16)
            v = cb_v[sl]
            cb_v[sl] = jnp.where(v > 0.0, 1.0 / jnp.maximum(v, 1.0), 0.0)
            return 0
        lax.fori_loop(0, (SEG_T + 8) // 16, inv_body, 0)
        pltpu.sync_copy(cb_v.at[pl.ds(0, SEG_T)],
                        inv_hbm.at[pl.ds(s * SEG_T, SEG_T)])


@functools.partial(
    pl.kernel,
    out_type=jax.ShapeDtypeStruct((NC, N, D), jnp.float32),
    mesh=_mesh,
    compiler_params=_sc_params_edge,
    scratch_types=[
        pltpu.VMEM((EB,), jnp.int32),
        pltpu.VMEM((EB,), jnp.int32),
        pltpu.VMEM((EB,), jnp.int32),
        pltpu.VMEM((EB,), jnp.int32),
        pltpu.VMEM((EB,), jnp.int32),
        pltpu.VMEM((EB,), jnp.int32),
        pltpu.VMEM((EB,), jnp.int32),
        pltpu.VMEM((EB,), jnp.float32),
        pltpu.VMEM((K, D // 2), jnp.int32),
        pltpu.VMEM((K, D // 2), jnp.int32),
        pltpu.VMEM((K, D), jnp.float32),
        pltpu.VMEM_SHARED((NR,), jnp.float32),
        pltpu.VMEM_SHARED((N, D), jnp.float32),
        pltpu.SemaphoreType.DMA,
        pltpu.SemaphoreType.DMA,
        pltpu.SemaphoreType.DMA,
        pltpu.SemaphoreType.DMA,
    ],
)
def _edge_pass_a(src_hbm, dst_hbm, et_hbm, inv_hbm, table_hbm, zeros_hbm,
                 out_hbm, et0, et1, ridx0, ridx1, dst0, dst1, seg_v, w_v,
                 rows0, rows1, sbuf, inv_sh, acc_sh, sem0, sem1, semb,
                 semw):
    c = lax.axis_index("c")
    s = lax.axis_index("s")

    base = (c * NS + s) * EW
    et_vs = (et0, et1)
    ridx_vs = (ridx0, ridx1)
    dst_vs = (dst0, dst1)

    def issue_block_loads(b):
        off = base + b * EB
        p = b % 2
        pltpu.async_copy(src_hbm.at[pl.ds(off, EB)], ridx_vs[p], semb)
        pltpu.async_copy(dst_hbm.at[pl.ds(off, EB)], dst_vs[p], semb)
        pltpu.async_copy(et_hbm.at[pl.ds(off, EB)], et_vs[p], semb)

    def drain_block_loads(b):
        p = b % 2
        pltpu.make_async_copy(src_hbm.at[pl.ds(0, EB)], ridx_vs[p],
                              semb).wait()
        pltpu.make_async_copy(dst_hbm.at[pl.ds(0, EB)], dst_vs[p],
                              semb).wait()
        pltpu.make_async_copy(et_hbm.at[pl.ds(0, EB)], et_vs[p],
                              semb).wait()

    issue_block_loads(0)

    @pl.when(s == 0)
    def _():
        pltpu.sync_copy(zeros_hbm, acc_sh)

    @pl.when(s == 1)
    def _():
        pltpu.sync_copy(inv_hbm, inv_sh)
    plsc.subcore_barrier()

    for b in range(NBLK):
        p = b % 2
        et_v = et_vs[p]
        ridx_v = ridx_vs[p]
        dst_v = dst_vs[p]
        drain_block_loads(b)

        def seg_i(i, _):
            sl = pl.ds(i * 16, 16)
            seg_v[sl] = dst_v[sl] * R + et_v[sl]
            ridx_v[sl] = et_v[sl] * N + ridx_v[sl]
            return 0
        lax.fori_loop(0, EB // 16, seg_i, 0)

        def wfire(q, _):
            pltpu.async_copy(inv_sh.at[seg_v.at[pl.ds(q * K, K)]],
                             w_v.at[pl.ds(q * K, K)], semw)
            return 0
        lax.fori_loop(0, CB, wfire, 0)

        def wdrain(q, _):
            pltpu.make_async_copy(inv_sh.at[seg_v.at[pl.ds(0, K)]],
                                  w_v.at[pl.ds(0, K)], semw).wait()
            return 0
        lax.fori_loop(0, CB, wdrain, 0)
        if b + 1 < NBLK:
            issue_block_loads(b + 1)

        def gather(q, buf, sem):
            pltpu.async_copy(table_hbm.at[ridx_v.at[pl.ds(q * K, K)]],
                             buf, sem)

        def step(q, buf, sem):
            pltpu.make_async_copy(table_hbm.at[ridx_v.at[pl.ds(0, K)]],
                                  buf, sem).wait()
            _scale_chunk_bf(w_v, q * K, buf, sbuf)
            pltpu.sync_copy(sbuf, acc_sh.at[dst_v.at[pl.ds(q * K, K)]],
                            add=True)

            @pl.when(q + 2 <= CB - 1)
            def _():
                gather(q + 2, buf, sem)

        gather(0, rows0, sem0)
        gather(1, rows1, sem1)

        def pairs(i, _):
            step(2 * i, rows0, sem0)
            step(2 * i + 1, rows1, sem1)
            return 0
        lax.fori_loop(0, (CB - 1) // 2, pairs, 0)
        step(CB - 1, rows0, sem0)

    plsc.subcore_barrier()

    @pl.when(s == 0)
    def _():
        pltpu.sync_copy(acc_sh, out_hbm.at[c])


@functools.partial(
    pl.kernel,
    out_type=jax.ShapeDtypeStruct((NC, N, D), jnp.float32),
    mesh=_mesh,
    compiler_params=_sc_params_edge,
    scratch_types=[
        pltpu.VMEM((EB,), jnp.int32),
        pltpu.VMEM((EB,), jnp.int32),
        pltpu.VMEM((EB,), jnp.int32),
        pltpu.VMEM((EB,), jnp.int32),
        pltpu.VMEM((EB,), jnp.float32),
        pltpu.VMEM((EB,), jnp.float32),
        pltpu.VMEM((K, D // 2), jnp.int32),
        pltpu.VMEM((K, D // 2), jnp.int32),
        pltpu.VMEM((K, D), jnp.float32),
        pltpu.VMEM_SHARED((N, D), jnp.float32),
        pltpu.SemaphoreType.DMA,
        pltpu.SemaphoreType.DMA,
        pltpu.SemaphoreType.DMA,
    ],
)
def _edge_pass_b(src_hbm, dst_hbm, w_hbm, table_hbm, zeros_hbm, out_hbm,
                 src0, src1, dst0, dst1, w0, w1, rows0, rows1, sbuf,
                 acc_sh, sem0, sem1, semb):
    c = lax.axis_index("c")
    s = lax.axis_index("s")
    base = (c * NS + s) * EW
    src_vs = (src0, src1)
    dst_vs = (dst0, dst1)
    w_vs = (w0, w1)

    def issue_block_loads(b):
        off = base + b * EB
        p = b % 2
        pltpu.async_copy(src_hbm.at[pl.ds(off, EB)], src_vs[p], semb)
        pltpu.async_copy(dst_hbm.at[pl.ds(off, EB)], dst_vs[p], semb)
        pltpu.async_copy(w_hbm.at[pl.ds(off, EB)], w_vs[p], semb)

    def drain_block_loads(b):
        p = b % 2
        pltpu.make_async_copy(src_hbm.at[pl.ds(0, EB)], src_vs[p],
                              semb).wait()
        pltpu.make_async_copy(dst_hbm.at[pl.ds(0, EB)], dst_vs[p],
                              semb).wait()
        pltpu.make_async_copy(w_hbm.at[pl.ds(0, EB)], w_vs[p],
                              semb).wait()

    issue_block_loads(0)

    @pl.when(s == 0)
    def _():
        pltpu.sync_copy(zeros_hbm, acc_sh)
    plsc.subcore_barrier()

    for b in range(NBLK):
        p = b % 2
        src_v = src_vs[p]
        dst_v = dst_vs[p]
        w_v = w_vs[p]
        drain_block_loads(b)
        if b + 1 < NBLK:
            issue_block_loads(b + 1)

        def gather(q, buf, sem):
            pltpu.async_copy(table_hbm.at[src_v.at[pl.ds(q * K, K)]],
                             buf, sem)

        def step(q, buf, sem):
            pltpu.make_async_copy(table_hbm.at[src_v.at[pl.ds(0, K)]],
                                  buf, sem).wait()
            _scale_chunk_bf(w_v, q * K, buf, sbuf)
            pltpu.sync_copy(sbuf, acc_sh.at[dst_v.at[pl.ds(q * K, K)]],
                            add=True)

            @pl.when(q + 2 <= CB - 1)
            def _():
                gather(q + 2, buf, sem)

        gather(0, rows0, sem0)
        gather(1, rows1, sem1)

        def pairs(i, _):
            step(2 * i, rows0, sem0)
            step(2 * i + 1, rows1, sem1)
            return 0
        lax.fori_loop(0, (CB - 1) // 2, pairs, 0)
        step(CB - 1, rows0, sem0)

    plsc.subcore_barrier()

    @pl.when(s == 0)
    def _():
        pltpu.sync_copy(acc_sh, out_hbm.at[c])


def _w_body(comp_ref, basis_ref, w_ref):
    r = pl.program_id(0)
    acc = basis_ref[0] * comp_ref[r, 0]
    for b in range(1, B):
        acc = acc + basis_ref[b] * comp_ref[r, b]
    w_ref[0] = acc


def _w_kernel(comp, basis):
    return pl.pallas_call(
        _w_body,
        grid=(R,),
        in_specs=[
            pl.BlockSpec(memory_space=pltpu.SMEM),
            pl.BlockSpec((B, D, H1), lambda r: (0, 0, 0)),
        ],
        out_specs=pl.BlockSpec((1, D, H1), lambda r: (r, 0, 0)),
        out_shape=jax.ShapeDtypeStruct((R, D, H1), jnp.float32),
    )(comp, basis)


def _h_body(x_ref, w_ref, h_ref):
    h_ref[...] = jnp.dot(x_ref[...], w_ref[0],
                         preferred_element_type=jnp.float32
                         ).astype(jnp.bfloat16)


def _h_kernel(x, w):
    return pl.pallas_call(
        _h_body,
        grid=(NB, R),
        in_specs=[
            pl.BlockSpec((BN, D), lambda i, r: (i, 0)),
            pl.BlockSpec((1, D, H1), lambda i, r: (r, 0, 0)),
        ],
        out_specs=pl.BlockSpec((BN, H1), lambda i, r: (r * NB + i, 0)),
        out_shape=jax.ShapeDtypeStruct((R * N, H1), jnp.bfloat16),
    )(x, w)


def _x1_body(agg_ref, x_ref, rw_ref, b_ref, x1_ref, x1bf_ref):
    x1 = (agg_ref[0] + agg_ref[1]
          + jnp.dot(x_ref[...], rw_ref[...],
                    preferred_element_type=jnp.float32)
          + b_ref[...])
    x1_ref[...] = x1
    t = lax.broadcasted_iota(jnp.int32, (BN, H1), 1)
    idx = (t // 32) * 32 + (t % 2) * 16 + (t % 32) // 2
    x1bf_ref[...] = jnp.take_along_axis(x1, idx, axis=1
                                        ).astype(jnp.bfloat16)


def _x1_kernel(agg2, x, root_w, bias1):
    return pl.pallas_call(
        _x1_body,
        grid=(NB,),
        in_specs=[
            pl.BlockSpec((NC, BN, H1), lambda i: (0, i, 0)),
            pl.BlockSpec((BN, D), lambda i: (i, 0)),
            pl.BlockSpec((D, H1), lambda i: (0, 0)),
            pl.BlockSpec((1, H1), lambda i: (0, 0)),
        ],
        out_specs=[pl.BlockSpec((BN, H1), lambda i: (i, 0)),
                   pl.BlockSpec((BN, H1), lambda i: (i, 0))],
        out_shape=[jax.ShapeDtypeStruct((N, H1), jnp.float32),
                   jax.ShapeDtypeStruct((N, H1), jnp.bfloat16)],
    )(agg2, x, root_w, bias1)


def _out_body(agg_ref, x1_ref, wr_ref, wr2_ref, b_ref, o_ref):
    o_ref[...] = (jnp.dot(agg_ref[0] + agg_ref[1], wr_ref[...],
                          preferred_element_type=jnp.float32)
                  + jnp.dot(x1_ref[...], wr2_ref[...],
                            preferred_element_type=jnp.float32)
                  + b_ref[...])


def _out_kernel(agg2, x1, w_rel, w_root2, bias2):
    return pl.pallas_call(
        _out_body,
        grid=(NB,),
        in_specs=[
            pl.BlockSpec((NC, BN, H1), lambda i: (0, i, 0)),
            pl.BlockSpec((BN, H1), lambda i: (i, 0)),
            pl.BlockSpec((H1, H2), lambda i: (0, 0)),
            pl.BlockSpec((H1, H2), lambda i: (0, 0)),
            pl.BlockSpec((1, H2), lambda i: (0, 0)),
        ],
        out_specs=pl.BlockSpec((BN, H2), lambda i: (i, 0)),
        out_shape=jax.ShapeDtypeStruct((N, H2), jnp.float32),
    )(agg2, x1, w_rel, w_root2, bias2)


def kernel(node_features, edge_index, edge_norm, edge_type, basis, comp,
           root_w, bias1, w_rel, w_root2, bias2):
    src = edge_index[0]
    dst = edge_index[1]
    et = edge_type
    en = edge_norm
    zeros_nd = jnp.zeros((N, D), jnp.float32)

    inv = _inv_table_kernel(dst, et)
    basis_p = basis[:, :, jnp.array(_PERM)]
    w_all = _w_kernel(comp, basis_p)
    h = _h_kernel(node_features, w_all)
    h32 = lax.bitcast_convert_type(h.reshape(R * N, D // 2, 2),
                                   jnp.int32)
    agg = _edge_pass_a(src, dst, et, inv, h32, zeros_nd)
    x1, x1bf = _x1_kernel(agg, node_features, root_w, bias1.reshape(1, H1))
    x1bf32 = lax.bitcast_convert_type(x1bf.reshape(N, H1 // 2, 2),
                                      jnp.int32)
    gg2 = _edge_pass_b(src, dst, en, x1bf32, zeros_nd)
    return _out_kernel(gg2, x1, w_rel, w_root2, bias2.reshape(1, H2))


# speedup vs baseline: 22.5983x; 2.3995x over previous
import functools

import jax
import jax.numpy as jnp
from jax import lax
from jax.experimental import pallas as pl
from jax.experimental.pallas import tpu as pltpu
from jax.experimental.pallas import tpu_sc as plsc

N = 10000
E = 320000
R = 8
B = 30
D = 128
H1 = 128
H2 = 128
NR = N * R

NC = 2
NS = 16
NW = NC * NS
EW = E // NW
K = 80
CH = EW // K

NB = 10
BN = N // NB

_mesh = plsc.VectorSubcoreMesh(
    core_axis_name="c", subcore_axis_name="s", num_cores=NC, num_subcores=NS)
_sc_params = pltpu.CompilerParams(needs_layout_passes=False)


EB = 2000
CB = EB // K
NBLK = EW // EB
ET2 = E // NS
NBLK2 = ET2 // EB
SEG_T = NR // NS


def _bcast16(vec, j):
    return jnp.take_along_axis(vec, jnp.full((16,), j, jnp.int32), axis=0)


def _scale_chunk(w_v, woff, buf):
    def scale_g(g, _):
        w16 = w_v[pl.ds(woff + g * 16, 16)]

        def scale_j4(j4, _):
            for u in range(4):
                jj = j4 * 4 + u
                wb = _bcast16(w16, jj)
                row = g * 16 + jj
                for col in range(D // 16):
                    cs = pl.ds(col * 16, 16)
                    buf[row, cs] = buf[row, cs] * wb
            return 0
        lax.fori_loop(0, 4, scale_j4, 0)
        return 0
    lax.fori_loop(0, K // 16, scale_g, 0)


@functools.partial(
    pl.kernel,
    out_type=jax.ShapeDtypeStruct((NR,), jnp.float32),
    mesh=_mesh,
    compiler_params=_sc_params,
    scratch_types=[
        pltpu.VMEM((EB,), jnp.int32),
        pltpu.VMEM((EB,), jnp.int32),
        pltpu.VMEM((EB,), jnp.int32),
        pltpu.VMEM((K,), jnp.float32),
        pltpu.VMEM((SEG_T + 8,), jnp.float32),
        pltpu.VMEM_SHARED((NR,), jnp.float32),
        pltpu.SemaphoreType.DMA,
    ],
)
def _inv_table_kernel(dst_hbm, et_hbm, inv_hbm, dst_v, et_v, seg_v, ones_v,
                      cb_v, cnt_sh, sem):
    c = lax.axis_index("c")
    s = lax.axis_index("s")

    def zfill(i, _):
        cb_v[pl.ds(i * 16, 16)] = jnp.zeros((16,), jnp.float32)
        return 0
    lax.fori_loop(0, (SEG_T + 8) // 16, zfill, 0)
    for g in range(K // 16):
        ones_v[pl.ds(g * 16, 16)] = jnp.ones((16,), jnp.float32)
    pltpu.sync_copy(cb_v.at[pl.ds(0, SEG_T)],
                    cnt_sh.at[pl.ds(s * SEG_T, SEG_T)])
    plsc.subcore_barrier()

    def count_block(b, _):
        off = s * ET2 + b * EB
        pltpu.sync_copy(dst_hbm.at[pl.ds(off, EB)], dst_v)
        pltpu.sync_copy(et_hbm.at[pl.ds(off, EB)], et_v)

        def seg_i(i, _):
            sl = pl.ds(i * 16, 16)
            seg_v[sl] = dst_v[sl] * R + et_v[sl]
            return 0
        lax.fori_loop(0, EB // 16, seg_i, 0)

        def fire(q, _):
            pltpu.async_copy(ones_v, cnt_sh.at[seg_v.at[pl.ds(q * K, K)]],
                             sem, add=True)
            return 0
        lax.fori_loop(0, CB, fire, 0)

        def drain(q, _):
            pltpu.make_async_copy(ones_v, cnt_sh.at[seg_v.at[pl.ds(0, K)]],
                                  sem).wait()
            return 0
        lax.fori_loop(0, CB, drain, 0)
        return 0
    lax.fori_loop(0, NBLK2, count_block, 0)
    plsc.subcore_barrier()

    @pl.when(c == 0)
    def _():
        pltpu.sync_copy(cnt_sh.at[pl.ds(s * SEG_T, SEG_T)],
                        cb_v.at[pl.ds(0, SEG_T)])

        def inv_body(i, _):
            sl = pl.ds(i * 16, 16)
            v = cb_v[sl]
            cb_v[sl] = jnp.where(v > 0.0, 1.0 / jnp.maximum(v, 1.0), 0.0)
            return 0
        lax.fori_loop(0, (SEG_T + 8) // 16, inv_body, 0)
        pltpu.sync_copy(cb_v.at[pl.ds(0, SEG_T)],
                        inv_hbm.at[pl.ds(s * SEG_T, SEG_T)])


@functools.partial(
    pl.kernel,
    out_type=jax.ShapeDtypeStruct((NC, N, D), jnp.float32),
    mesh=_mesh,
    compiler_params=_sc_params,
    scratch_types=[
        pltpu.VMEM((EB,), jnp.int32),
        pltpu.VMEM((EB,), jnp.int32),
        pltpu.VMEM((EB,), jnp.int32),
        pltpu.VMEM((EB,), jnp.int32),
        pltpu.VMEM((EB,), jnp.int32),
        pltpu.VMEM((EB,), jnp.int32),
        pltpu.VMEM((EB,), jnp.int32),
        pltpu.VMEM((EB,), jnp.float32),
        pltpu.VMEM((K, D), jnp.float32),
        pltpu.VMEM((K, D), jnp.float32),
        pltpu.VMEM_SHARED((NR,), jnp.float32),
        pltpu.VMEM_SHARED((N, D), jnp.float32),
        pltpu.SemaphoreType.DMA,
        pltpu.SemaphoreType.DMA,
        pltpu.SemaphoreType.DMA,
        pltpu.SemaphoreType.DMA,
    ],
)
def _edge_pass_a(src_hbm, dst_hbm, et_hbm, inv_hbm, table_hbm, zeros_hbm,
                 out_hbm, et0, et1, ridx0, ridx1, dst0, dst1, seg_v, w_v,
                 rows0, rows1, inv_sh, acc_sh, sem0, sem1, semb, semw):
    c = lax.axis_index("c")
    s = lax.axis_index("s")

    base = (c * NS + s) * EW
    et_vs = (et0, et1)
    ridx_vs = (ridx0, ridx1)
    dst_vs = (dst0, dst1)

    def issue_block_loads(b):
        off = base + b * EB
        p = b % 2
        pltpu.async_copy(src_hbm.at[pl.ds(off, EB)], ridx_vs[p], semb)
        pltpu.async_copy(dst_hbm.at[pl.ds(off, EB)], dst_vs[p], semb)
        pltpu.async_copy(et_hbm.at[pl.ds(off, EB)], et_vs[p], semb)

    def drain_block_loads(b):
        p = b % 2
        pltpu.make_async_copy(src_hbm.at[pl.ds(0, EB)], ridx_vs[p],
                              semb).wait()
        pltpu.make_async_copy(dst_hbm.at[pl.ds(0, EB)], dst_vs[p],
                              semb).wait()
        pltpu.make_async_copy(et_hbm.at[pl.ds(0, EB)], et_vs[p],
                              semb).wait()

    issue_block_loads(0)

    @pl.when(s == 0)
    def _():
        pltpu.sync_copy(zeros_hbm, acc_sh)

    @pl.when(s == 1)
    def _():
        pltpu.sync_copy(inv_hbm, inv_sh)
    plsc.subcore_barrier()

    for b in range(NBLK):
        p = b % 2
        et_v = et_vs[p]
        ridx_v = ridx_vs[p]
        dst_v = dst_vs[p]
        drain_block_loads(b)

        def seg_i(i, _):
            sl = pl.ds(i * 16, 16)
            seg_v[sl] = dst_v[sl] * R + et_v[sl]
            ridx_v[sl] = et_v[sl] * N + ridx_v[sl]
            return 0
        lax.fori_loop(0, EB // 16, seg_i, 0)

        def wfire(q, _):
            pltpu.async_copy(inv_sh.at[seg_v.at[pl.ds(q * K, K)]],
                             w_v.at[pl.ds(q * K, K)], semw)
            return 0
        lax.fori_loop(0, CB, wfire, 0)

        def wdrain(q, _):
            pltpu.make_async_copy(inv_sh.at[seg_v.at[pl.ds(0, K)]],
                                  w_v.at[pl.ds(0, K)], semw).wait()
            return 0
        lax.fori_loop(0, CB, wdrain, 0)
        if b + 1 < NBLK:
            issue_block_loads(b + 1)

        def gather(q, buf, sem):
            pltpu.async_copy(table_hbm.at[ridx_v.at[pl.ds(q * K, K)]],
                             buf, sem)

        def step(q, buf, sem):
            pltpu.make_async_copy(table_hbm.at[ridx_v.at[pl.ds(0, K)]],
                                  buf, sem).wait()
            _scale_chunk(w_v, q * K, buf)
            pltpu.sync_copy(buf, acc_sh.at[dst_v.at[pl.ds(q * K, K)]],
                            add=True)

            @pl.when(q + 2 <= CB - 1)
            def _():
                gather(q + 2, buf, sem)

        gather(0, rows0, sem0)
        gather(1, rows1, sem1)

        def pairs(i, _):
            step(2 * i, rows0, sem0)
            step(2 * i + 1, rows1, sem1)
            return 0
        lax.fori_loop(0, (CB - 1) // 2, pairs, 0)
        step(CB - 1, rows0, sem0)

    plsc.subcore_barrier()

    @pl.when(s == 0)
    def _():
        pltpu.sync_copy(acc_sh, out_hbm.at[c])


@functools.partial(
    pl.kernel,
    out_type=jax.ShapeDtypeStruct((NC, N, D), jnp.float32),
    mesh=_mesh,
    compiler_params=_sc_params,
    scratch_types=[
        pltpu.VMEM((EB,), jnp.int32),
        pltpu.VMEM((EB,), jnp.int32),
        pltpu.VMEM((EB,), jnp.int32),
        pltpu.VMEM((EB,), jnp.int32),
        pltpu.VMEM((EB,), jnp.float32),
        pltpu.VMEM((EB,), jnp.float32),
        pltpu.VMEM((K, D), jnp.float32),
        pltpu.VMEM((K, D), jnp.float32),
        pltpu.VMEM_SHARED((N, D), jnp.float32),
        pltpu.SemaphoreType.DMA,
        pltpu.SemaphoreType.DMA,
        pltpu.SemaphoreType.DMA,
    ],
)
def _edge_pass_b(src_hbm, dst_hbm, w_hbm, table_hbm, zeros_hbm, out_hbm,
                 src0, src1, dst0, dst1, w0, w1, rows0, rows1, acc_sh,
                 sem0, sem1, semb):
    c = lax.axis_index("c")
    s = lax.axis_index("s")
    base = (c * NS + s) * EW
    src_vs = (src0, src1)
    dst_vs = (dst0, dst1)
    w_vs = (w0, w1)

    def issue_block_loads(b):
        off = base + b * EB
        p = b % 2
        pltpu.async_copy(src_hbm.at[pl.ds(off, EB)], src_vs[p], semb)
        pltpu.async_copy(dst_hbm.at[pl.ds(off, EB)], dst_vs[p], semb)
        pltpu.async_copy(w_hbm.at[pl.ds(off, EB)], w_vs[p], semb)

    def drain_block_loads(b):
        p = b % 2
        pltpu.make_async_copy(src_hbm.at[pl.ds(0, EB)], src_vs[p],
                              semb).wait()
        pltpu.make_async_copy(dst_hbm.at[pl.ds(0, EB)], dst_vs[p],
                              semb).wait()
        pltpu.make_async_copy(w_hbm.at[pl.ds(0, EB)], w_vs[p],
                              semb).wait()

    issue_block_loads(0)

    @pl.when(s == 0)
    def _():
        pltpu.sync_copy(zeros_hbm, acc_sh)
    plsc.subcore_barrier()

    for b in range(NBLK):
        p = b % 2
        src_v = src_vs[p]
        dst_v = dst_vs[p]
        w_v = w_vs[p]
        drain_block_loads(b)
        if b + 1 < NBLK:
            issue_block_loads(b + 1)

        def gather(q, buf, sem):
            pltpu.async_copy(table_hbm.at[src_v.at[pl.ds(q * K, K)]],
                             buf, sem)

        def step(q, buf, sem):
            pltpu.make_async_copy(table_hbm.at[src_v.at[pl.ds(0, K)]],
                                  buf, sem).wait()
            _scale_chunk(w_v, q * K, buf)
            pltpu.sync_copy(buf, acc_sh.at[dst_v.at[pl.ds(q * K, K)]],
                            add=True)

            @pl.when(q + 2 <= CB - 1)
            def _():
                gather(q + 2, buf, sem)

        gather(0, rows0, sem0)
        gather(1, rows1, sem1)

        def pairs(i, _):
            step(2 * i, rows0, sem0)
            step(2 * i + 1, rows1, sem1)
            return 0
        lax.fori_loop(0, (CB - 1) // 2, pairs, 0)
        step(CB - 1, rows0, sem0)

    plsc.subcore_barrier()

    @pl.when(s == 0)
    def _():
        pltpu.sync_copy(acc_sh, out_hbm.at[c])


def _w_body(comp_ref, basis_ref, w_ref):
    r = pl.program_id(0)
    acc = basis_ref[0] * comp_ref[r, 0]
    for b in range(1, B):
        acc = acc + basis_ref[b] * comp_ref[r, b]
    w_ref[0] = acc


def _w_kernel(comp, basis):
    return pl.pallas_call(
        _w_body,
        grid=(R,),
        in_specs=[
            pl.BlockSpec(memory_space=pltpu.SMEM),
            pl.BlockSpec((B, D, H1), lambda r: (0, 0, 0)),
        ],
        out_specs=pl.BlockSpec((1, D, H1), lambda r: (r, 0, 0)),
        out_shape=jax.ShapeDtypeStruct((R, D, H1), jnp.float32),
    )(comp, basis)


def _h_body(x_ref, w_ref, h_ref):
    h_ref[...] = jnp.dot(x_ref[...], w_ref[0],
                         preferred_element_type=jnp.float32)


def _h_kernel(x, w):
    return pl.pallas_call(
        _h_body,
        grid=(NB, R),
        in_specs=[
            pl.BlockSpec((BN, D), lambda i, r: (i, 0)),
            pl.BlockSpec((1, D, H1), lambda i, r: (r, 0, 0)),
        ],
        out_specs=pl.BlockSpec((BN, H1), lambda i, r: (r * NB + i, 0)),
        out_shape=jax.ShapeDtypeStruct((R * N, H1), jnp.float32),
    )(x, w)


def _x1_body(agg_ref, x_ref, rw_ref, b_ref, x1_ref):
    x1_ref[...] = (agg_ref[0] + agg_ref[1]
                   + jnp.dot(x_ref[...], rw_ref[...],
                             preferred_element_type=jnp.float32)
                   + b_ref[...])


def _x1_kernel(agg2, x, root_w, bias1):
    return pl.pallas_call(
        _x1_body,
        grid=(NB,),
        in_specs=[
            pl.BlockSpec((NC, BN, H1), lambda i: (0, i, 0)),
            pl.BlockSpec((BN, D), lambda i: (i, 0)),
            pl.BlockSpec((D, H1), lambda i: (0, 0)),
            pl.BlockSpec((1, H1), lambda i: (0, 0)),
        ],
        out_specs=pl.BlockSpec((BN, H1), lambda i: (i, 0)),
        out_shape=jax.ShapeDtypeStruct((N, H1), jnp.float32),
    )(agg2, x, root_w, bias1)


def _out_body(agg_ref, x1_ref, wr_ref, wr2_ref, b_ref, o_ref):
    o_ref[...] = (jnp.dot(agg_ref[0] + agg_ref[1], wr_ref[...],
                          preferred_element_type=jnp.float32)
                  + jnp.dot(x1_ref[...], wr2_ref[...],
                            preferred_element_type=jnp.float32)
                  + b_ref[...])


def _out_kernel(agg2, x1, w_rel, w_root2, bias2):
    return pl.pallas_call(
        _out_body,
        grid=(NB,),
        in_specs=[
            pl.BlockSpec((NC, BN, H1), lambda i: (0, i, 0)),
            pl.BlockSpec((BN, H1), lambda i: (i, 0)),
            pl.BlockSpec((H1, H2), lambda i: (0, 0)),
            pl.BlockSpec((H1, H2), lambda i: (0, 0)),
            pl.BlockSpec((1, H2), lambda i: (0, 0)),
        ],
        out_specs=pl.BlockSpec((BN, H2), lambda i: (i, 0)),
        out_shape=jax.ShapeDtypeStruct((N, H2), jnp.float32),
    )(agg2, x1, w_rel, w_root2, bias2)


def kernel(node_features, edge_index, edge_norm, edge_type, basis, comp,
           root_w, bias1, w_rel, w_root2, bias2):
    src = edge_index[0]
    dst = edge_index[1]
    et = edge_type
    en = edge_norm
    zeros_nd = jnp.zeros((N, D), jnp.float32)

    inv = _inv_table_kernel(dst, et)
    w_all = _w_kernel(comp, basis)
    h = _h_kernel(node_features, w_all)
    agg = _edge_pass_a(src, dst, et, inv, h, zeros_nd)
    x1 = _x1_kernel(agg, node_features, root_w, bias1.reshape(1, H1))
    gg2 = _edge_pass_b(src, dst, en, x1, zeros_nd)
    return _out_kernel(gg2, x1, w_rel, w_root2, bias2.reshape(1, H2))
